# baseline (speedup 1.0000x reference)
import functools

import jax
import jax.numpy as jnp
from jax.experimental import pallas as pl
from jax.experimental.pallas import tpu as pltpu

C_PAD = 128
S = 8
CSLOT = 16
NG = 8
N_CLS = 5


def _round_up(v, m):
    return (v + m - 1) // m * m


def _bdims(T, K0, stride0, poolk_a, pools_a, K3, poolk_b, pools_b):
    pad0_l = K0 // 2 + (K0 % 2) - 1
    pad0_r = K0 // 2
    Hp = T + pad0_l + pad0_r
    L0 = (Hp - K0) // stride0 + 1
    Ks0 = -(-K0 // stride0)
    L1 = (L0 - poolk_a) // pools_a + 1
    pad3 = K3 // 2 + (K3 % 2) - 1
    L2 = (L1 - poolk_b) // pools_b + 1
    return dict(K0=K0, stride0=stride0, pad0_l=pad0_l, pad0_r=pad0_r,
                L0=L0, Ks0=Ks0,
                poolk_a=poolk_a, pools_a=pools_a, L1=L1,
                K3=K3, pad3=pad3,
                poolk_b=poolk_b, pools_b=pools_b, L2=L2)


def _packed_kernel(xn_ref,
                   w01_ref, wm1_ref, bb1_ref,
                   w02_ref, wm2_ref, bb2_ref,
                   cls_ref, o_ref,
                   xts_g, buf0_g, buf1_g, buf2_g, *, d1, d2, tp, mp):
    f32 = jnp.float32

    ri = jax.lax.broadcasted_iota(jnp.int32, (S, C_PAD), 0)
    ci = jax.lax.broadcasted_iota(jnp.int32, (S, C_PAD), 1)
    ident = (ri == ci).astype(f32)
    t_len = xn_ref.shape[2]

    for g in range(NG):
        xts = xts_g[g]
        xts[pl.ds(0, mp), :] = jnp.zeros((mp, C_PAD), f32)
        xts[pl.ds(mp + t_len, tp - mp - t_len), :] = jnp.zeros(
            (tp - mp - t_len, C_PAD), f32)
        xts[pl.ds(mp, t_len), :] = jax.lax.dot_general(
            xn_ref[g], ident, (((0,), (0,)), ((), ())),
            preferred_element_type=f32)

    def run_branch(w0_ref, wm_ref, bb_ref, d):
        L0, K0, st, off = d["L0"], d["K0"], d["stride0"], d["row_off"]
        poolk_a, pools_a, L1 = d["poolk_a"], d["pools_a"], d["L1"]
        K3, pad3 = d["K3"], d["pad3"]
        poolk_b, pools_b, L2 = d["poolk_b"], d["pools_b"], d["L2"]
        hi_pad = K3 - 1 - pad3

        for g in range(NG):
            for buf in (buf1_g[g], buf2_g[g]):
                buf[pl.ds(0, pad3), :] = jnp.zeros((pad3, C_PAD), f32)
                buf[pl.ds(pad3 + L1, hi_pad), :] = jnp.zeros(
                    (hi_pad, C_PAD), f32)

        Ks0 = d["Ks0"]
        Lph = L0 + Ks0 - 1
        for g in range(NG):
            xts = xts_g[g]
            xi = xts[pl.ds(off, Lph, stride=st), :]
            for r in range(1, st):
                xi = xi + jnp.roll(xts[pl.ds(off + r, Lph, stride=st), :],
                                   r * S, axis=1)
            acc = jnp.dot(xi[0:L0, :], w0_ref[0], preferred_element_type=f32)
            for ks in range(1, Ks0):
                acc = acc + jnp.dot(xi[ks:ks + L0, :], w0_ref[ks],
                                    preferred_element_type=f32)
            buf0_g[g][pl.ds(0, L0), :] = acc

        for g in range(NG):
            buf0 = buf0_g[g]
            pooled = buf0[pl.ds(0, L1, stride=pools_a), :]
            for r in range(1, poolk_a):
                pooled = jnp.maximum(pooled,
                                     buf0[pl.ds(r, L1, stride=pools_a), :])
            buf1_g[g][pl.ds(pad3, L1), :] = jnp.maximum(
                pooled + bb_ref[pl.ds(0, 1), :], 0.0)

        def conv_same(src_ref, layer):
            a = jnp.dot(src_ref[pl.ds(0, L1), :], wm_ref[layer, 0],
                        preferred_element_type=f32)
            for k in range(1, K3):
                a = a + jnp.dot(src_ref[pl.ds(k, L1), :], wm_ref[layer, k],
                                preferred_element_type=f32)
            return jnp.maximum(a + bb_ref[pl.ds(layer + 1, 1), :], 0.0)

        for g in range(NG):
            buf2_g[g][pl.ds(pad3, L1), :] = conv_same(buf1_g[g], 0)
        for g in range(NG):
            buf1_g[g][pl.ds(pad3, L1), :] = conv_same(buf2_g[g], 1)
        for g in range(NG):
            buf0_g[g][pl.ds(0, L1), :] = conv_same(buf1_g[g], 2)

        outs = []
        for g in range(NG):
            buf0 = buf0_g[g]
            out = buf0[pl.ds(0, L2, stride=pools_b), :]
            for r in range(1, poolk_b):
                out = jnp.maximum(out, buf0[pl.ds(r, L2, stride=pools_b), :])
            outs.append(out)
        return outs

    o1s = run_branch(w01_ref, wm1_ref, bb1_ref, d1)
    o2s = run_branch(w02_ref, wm2_ref, bb2_ref, d2)

    L2_1, L2_2 = d1["L2"], d2["L2"]
    row = jax.lax.broadcasted_iota(jnp.int32, (C_PAD, C_PAD), 0)
    col = jax.lax.broadcasted_iota(jnp.int32, (C_PAD, C_PAD), 1)
    sel = ((row // CSLOT) == col).astype(f32)
    ri16 = jax.lax.broadcasted_iota(jnp.int32, (CSLOT, C_PAD), 0)
    ci16 = jax.lax.broadcasted_iota(jnp.int32, (CSLOT, C_PAD), 1)
    m16 = ((ci16 % CSLOT) == ri16).astype(f32)
    for g in range(NG):
        z = (jax.lax.dot_general(cls_ref[pl.ds(0, L2_1), :], o1s[g],
                                 (((0,), (0,)), ((), ())),
                                 preferred_element_type=f32)
             + jax.lax.dot_general(cls_ref[pl.ds(L2_1, L2_2), :], o2s[g],
                                   (((0,), (0,)), ((), ())),
                                   preferred_element_type=f32))
        rows = []
        for n in range(N_CLS):
            rows.append(jnp.sum(z[n * CSLOT:(n + 1) * CSLOT, :] * m16,
                                axis=0, keepdims=True))
        rows.append(jnp.zeros((S - N_CLS, C_PAD), f32))
        vmat = jnp.concatenate(rows, axis=0)
        o_ref[g] = jnp.dot(vmat, sel, preferred_element_type=f32)


def _blockdiag(w):
    return jnp.kron(jnp.eye(S, dtype=w.dtype), w)


def kernel(x, b1_w0r, b1_wmid, b1_biases, b2_w0r, b2_wmid, b2_biases,
           cls_wperm, cls_b):
    T = x.shape[2]
    d1 = _bdims(T, 8, 2, 2, 2, 4, 2, 2)
    d2 = _bdims(T, 16, 4, 2, 2, 4, 2, 2)
    B = x.shape[0]
    G = B // S
    L2_sum = d1["L2"] + d2["L2"]
    xs = x[:, 0, :, 0]

    mp = max(d1["pad0_l"], d2["pad0_l"])
    d1["row_off"] = mp - d1["pad0_l"]
    d2["row_off"] = mp - d2["pad0_l"]
    need = max(d["row_off"] + d["K0"] + (d["L0"] - 1) * d["stride0"]
               for d in (d1, d2))
    TP = _round_up(need, 8)
    xng = xs.reshape(G, S, T)

    def _tap_weights(w0r, d):
        ks0, st = d["Ks0"], d["stride0"]
        w0k = w0r.reshape(ks0 * st, C_PAD)[:, :CSLOT].reshape(ks0, st, CSLOT)
        eye = jnp.eye(S, dtype=w0k.dtype)
        e = (w0k[:, :, None, None, :]
             * eye[None, None, :, :, None]).reshape(ks0, st * S, C_PAD)
        return jnp.pad(e, ((0, 0), (0, C_PAD - st * S), (0, 0)))

    w0b1 = _tap_weights(b1_w0r, d1)
    w0b2 = _tap_weights(b2_w0r, d2)

    wmb1 = jax.vmap(jax.vmap(_blockdiag))(b1_wmid[:, :, :CSLOT, :CSLOT])
    wmb2 = jax.vmap(jax.vmap(_blockdiag))(b2_wmid[:, :, :CSLOT, :CSLOT])
    bb1 = jnp.tile(b1_biases[:, :CSLOT], (1, S))
    bb2 = jnp.tile(b2_biases[:, :CSLOT], (1, S))

    wc = cls_wperm.reshape(L2_sum, C_PAD, N_CLS)[:, :CSLOT, :]
    wc = jnp.transpose(wc, (0, 2, 1)).reshape(L2_sum, N_CLS * CSLOT)

    rows0 = _round_up(max(d1["L0"], d2["L0"]), 8)
    rows1 = _round_up(max(d1["L1"] + d1["K3"] - 1, d2["L1"] + d2["K3"] - 1), 8)

    kern = functools.partial(_packed_kernel, d1=d1, d2=d2, tp=TP, mp=mp)
    raw = pl.pallas_call(
        kern,
        out_shape=jax.ShapeDtypeStruct((G, S, C_PAD), jnp.float32),
        grid=(G // NG,),
        in_specs=[
            pl.BlockSpec((NG, S, T), lambda b: (b, 0, 0)),
            pl.BlockSpec((d1["Ks0"], C_PAD, C_PAD), lambda b: (0, 0, 0)),
            pl.BlockSpec((3, d1["K3"], C_PAD, C_PAD), lambda b: (0, 0, 0, 0)),
            pl.BlockSpec((4, C_PAD), lambda b: (0, 0)),
            pl.BlockSpec((d2["Ks0"], C_PAD, C_PAD), lambda b: (0, 0, 0)),
            pl.BlockSpec((3, d2["K3"], C_PAD, C_PAD), lambda b: (0, 0, 0, 0)),
            pl.BlockSpec((4, C_PAD), lambda b: (0, 0)),
            pl.BlockSpec((L2_sum, N_CLS * CSLOT), lambda b: (0, 0)),
        ],
        out_specs=pl.BlockSpec((NG, S, C_PAD), lambda b: (b, 0, 0)),
        scratch_shapes=[
            [pltpu.VMEM((TP, C_PAD), jnp.float32) for _ in range(NG)],
            [pltpu.VMEM((rows0, C_PAD), jnp.float32) for _ in range(NG)],
            [pltpu.VMEM((rows1, C_PAD), jnp.float32) for _ in range(NG)],
            [pltpu.VMEM((rows1, C_PAD), jnp.float32) for _ in range(NG)],
        ],
        compiler_params=pltpu.CompilerParams(
            dimension_semantics=("parallel",)),
    )(xng, w0b1, wmb1, bb1, w0b2, wmb2, bb2, wc)

    logits = jnp.transpose(raw[:, :N_CLS, :S], (0, 2, 1)).reshape(B, N_CLS)
    return logits + cls_b


# speedup vs baseline: 1.0445x; 1.0445x over previous
import functools

import jax
import jax.numpy as jnp
from jax.experimental import pallas as pl
from jax.experimental.pallas import tpu as pltpu

C_PAD = 128
S = 8
CSLOT = 16
NG = 8
N_CLS = 5


def _round_up(v, m):
    return (v + m - 1) // m * m


def _bdims(T, K0, stride0, poolk_a, pools_a, K3, poolk_b, pools_b):
    pad0_l = K0 // 2 + (K0 % 2) - 1
    pad0_r = K0 // 2
    Hp = T + pad0_l + pad0_r
    L0 = (Hp - K0) // stride0 + 1
    Ks0 = -(-K0 // stride0)
    L1 = (L0 - poolk_a) // pools_a + 1
    pad3 = K3 // 2 + (K3 % 2) - 1
    L2 = (L1 - poolk_b) // pools_b + 1
    return dict(K0=K0, stride0=stride0, pad0_l=pad0_l, pad0_r=pad0_r,
                L0=L0, Ks0=Ks0,
                poolk_a=poolk_a, pools_a=pools_a, L1=L1,
                K3=K3, pad3=pad3,
                poolk_b=poolk_b, pools_b=pools_b, L2=L2)


def _packed_kernel(xn_ref,
                   w01_ref, wm1_ref, bb1_ref,
                   w02_ref, wm2_ref, bb2_ref,
                   cls_ref, o_ref,
                   xts_g, buf0_g, buf1_g, buf2_g, *, d1, d2, tp, mp):
    f32 = jnp.float32

    ri = jax.lax.broadcasted_iota(jnp.int32, (S, C_PAD), 0)
    ci = jax.lax.broadcasted_iota(jnp.int32, (S, C_PAD), 1)
    ident = (ri == ci).astype(f32)
    t_len = xn_ref.shape[2]

    for g in range(NG):
        xts = xts_g[g]
        xts[pl.ds(0, mp), :] = jnp.zeros((mp, C_PAD), f32)
        xts[pl.ds(mp + t_len, tp - mp - t_len), :] = jnp.zeros(
            (tp - mp - t_len, C_PAD), f32)
        xts[pl.ds(mp, t_len), :] = jax.lax.dot_general(
            xn_ref[g], ident, (((0,), (0,)), ((), ())),
            preferred_element_type=f32)

    def run_branch(w0_ref, wm_ref, bb_ref, d):
        L0, K0, st, off = d["L0"], d["K0"], d["stride0"], d["row_off"]
        poolk_a, pools_a, L1 = d["poolk_a"], d["pools_a"], d["L1"]
        K3, pad3 = d["K3"], d["pad3"]
        poolk_b, pools_b, L2 = d["poolk_b"], d["pools_b"], d["L2"]
        hi_pad = K3 - 1 - pad3

        for g in range(NG):
            for buf in (buf1_g[g], buf2_g[g]):
                buf[pl.ds(0, pad3), :] = jnp.zeros((pad3, C_PAD), f32)
                buf[pl.ds(pad3 + L1, hi_pad), :] = jnp.zeros(
                    (hi_pad, C_PAD), f32)

        Ks0 = d["Ks0"]
        Lph = L0 + Ks0 - 1
        for g in range(NG):
            xts = xts_g[g]
            xi = xts[pl.ds(off, Lph, stride=st), :]
            for r in range(1, st):
                xi = xi + jnp.roll(xts[pl.ds(off + r, Lph, stride=st), :],
                                   r * S, axis=1)
            acc = jnp.dot(xi[0:L0, :], w0_ref[0], preferred_element_type=f32)
            for ks in range(1, Ks0):
                acc = acc + jnp.dot(xi[ks:ks + L0, :], w0_ref[ks],
                                    preferred_element_type=f32)
            buf0_g[g][pl.ds(0, L0), :] = acc

        for g in range(NG):
            buf0 = buf0_g[g]
            pooled = buf0[pl.ds(0, L1, stride=pools_a), :]
            for r in range(1, poolk_a):
                pooled = jnp.maximum(pooled,
                                     buf0[pl.ds(r, L1, stride=pools_a), :])
            buf1_g[g][pl.ds(pad3, L1), :] = jnp.maximum(
                pooled + bb_ref[pl.ds(0, 1), :], 0.0)

        def conv_same(src_ref, layer):
            a = jnp.dot(src_ref[pl.ds(0, L1), :], wm_ref[layer, 0],
                        preferred_element_type=f32)
            for k in range(1, K3):
                a = a + jnp.dot(src_ref[pl.ds(k, L1), :], wm_ref[layer, k],
                                preferred_element_type=f32)
            return jnp.maximum(a + bb_ref[pl.ds(layer + 1, 1), :], 0.0)

        for g in range(NG):
            buf2_g[g][pl.ds(pad3, L1), :] = conv_same(buf1_g[g], 0)
        for g in range(NG):
            buf1_g[g][pl.ds(pad3, L1), :] = conv_same(buf2_g[g], 1)
        for g in range(NG):
            buf0_g[g][pl.ds(0, L1), :] = conv_same(buf1_g[g], 2)

        outs = []
        for g in range(NG):
            buf0 = buf0_g[g]
            out = buf0[pl.ds(0, L2, stride=pools_b), :]
            for r in range(1, poolk_b):
                out = jnp.maximum(out, buf0[pl.ds(r, L2, stride=pools_b), :])
            outs.append(out)
        return outs

    o1s = run_branch(w01_ref, wm1_ref, bb1_ref, d1)
    o2s = run_branch(w02_ref, wm2_ref, bb2_ref, d2)

    L2_1, L2_2 = d1["L2"], d2["L2"]
    row = jax.lax.broadcasted_iota(jnp.int32, (C_PAD, C_PAD), 0)
    col = jax.lax.broadcasted_iota(jnp.int32, (C_PAD, C_PAD), 1)
    sel = ((row // CSLOT) == col).astype(f32)
    for g in range(NG):
        rows = []
        for n in range(N_CLS):
            rows.append(
                jnp.sum(o1s[g] * cls_ref[n, pl.ds(0, L2_1), :],
                        axis=0, keepdims=True)
                + jnp.sum(o2s[g] * cls_ref[n, pl.ds(L2_1, L2_2), :],
                          axis=0, keepdims=True))
        rows.append(jnp.zeros((S - N_CLS, C_PAD), f32))
        vmat = jnp.concatenate(rows, axis=0)
        o_ref[g] = jnp.dot(vmat, sel, preferred_element_type=f32)


def _blockdiag(w):
    return jnp.kron(jnp.eye(S, dtype=w.dtype), w)


def kernel(x, b1_w0r, b1_wmid, b1_biases, b2_w0r, b2_wmid, b2_biases,
           cls_wperm, cls_b):
    T = x.shape[2]
    d1 = _bdims(T, 8, 2, 2, 2, 4, 2, 2)
    d2 = _bdims(T, 16, 4, 2, 2, 4, 2, 2)
    B = x.shape[0]
    G = B // S
    L2_sum = d1["L2"] + d2["L2"]
    xs = x[:, 0, :, 0]

    mp = max(d1["pad0_l"], d2["pad0_l"])
    d1["row_off"] = mp - d1["pad0_l"]
    d2["row_off"] = mp - d2["pad0_l"]
    need = max(d["row_off"] + d["K0"] + (d["L0"] - 1) * d["stride0"]
               for d in (d1, d2))
    TP = _round_up(need, 8)
    xng = xs.reshape(G, S, T)

    def _tap_weights(w0r, d):
        ks0, st = d["Ks0"], d["stride0"]
        w0k = w0r.reshape(ks0 * st, C_PAD)[:, :CSLOT].reshape(ks0, st, CSLOT)
        eye = jnp.eye(S, dtype=w0k.dtype)
        e = (w0k[:, :, None, None, :]
             * eye[None, None, :, :, None]).reshape(ks0, st * S, C_PAD)
        return jnp.pad(e, ((0, 0), (0, C_PAD - st * S), (0, 0)))

    w0b1 = _tap_weights(b1_w0r, d1)
    w0b2 = _tap_weights(b2_w0r, d2)

    wmb1 = jax.vmap(jax.vmap(_blockdiag))(b1_wmid[:, :, :CSLOT, :CSLOT])
    wmb2 = jax.vmap(jax.vmap(_blockdiag))(b2_wmid[:, :, :CSLOT, :CSLOT])
    bb1 = jnp.tile(b1_biases[:, :CSLOT], (1, S))
    bb2 = jnp.tile(b2_biases[:, :CSLOT], (1, S))

    wc = cls_wperm.reshape(L2_sum, C_PAD, N_CLS)[:, :CSLOT, :]
    wc = jnp.tile(jnp.transpose(wc, (2, 0, 1)), (1, 1, S))
    wc = jnp.pad(wc, ((0, S - N_CLS), (0, 0), (0, 0)))

    rows0 = _round_up(max(d1["L0"], d2["L0"]), 8)
    rows1 = _round_up(max(d1["L1"] + d1["K3"] - 1, d2["L1"] + d2["K3"] - 1), 8)

    kern = functools.partial(_packed_kernel, d1=d1, d2=d2, tp=TP, mp=mp)
    raw = pl.pallas_call(
        kern,
        out_shape=jax.ShapeDtypeStruct((G, S, C_PAD), jnp.float32),
        grid=(G // NG,),
        in_specs=[
            pl.BlockSpec((NG, S, T), lambda b: (b, 0, 0)),
            pl.BlockSpec((d1["Ks0"], C_PAD, C_PAD), lambda b: (0, 0, 0)),
            pl.BlockSpec((3, d1["K3"], C_PAD, C_PAD), lambda b: (0, 0, 0, 0)),
            pl.BlockSpec((4, C_PAD), lambda b: (0, 0)),
            pl.BlockSpec((d2["Ks0"], C_PAD, C_PAD), lambda b: (0, 0, 0)),
            pl.BlockSpec((3, d2["K3"], C_PAD, C_PAD), lambda b: (0, 0, 0, 0)),
            pl.BlockSpec((4, C_PAD), lambda b: (0, 0)),
            pl.BlockSpec((S, L2_sum, C_PAD), lambda b: (0, 0, 0)),
        ],
        out_specs=pl.BlockSpec((NG, S, C_PAD), lambda b: (b, 0, 0)),
        scratch_shapes=[
            [pltpu.VMEM((TP, C_PAD), jnp.float32) for _ in range(NG)],
            [pltpu.VMEM((rows0, C_PAD), jnp.float32) for _ in range(NG)],
            [pltpu.VMEM((rows1, C_PAD), jnp.float32) for _ in range(NG)],
            [pltpu.VMEM((rows1, C_PAD), jnp.float32) for _ in range(NG)],
        ],
        compiler_params=pltpu.CompilerParams(
            dimension_semantics=("parallel",)),
    )(xng, w0b1, wmb1, bb1, w0b2, wmb2, bb2, wc)

    logits = jnp.transpose(raw[:, :N_CLS, :S], (0, 2, 1)).reshape(B, N_CLS)
    return logits + cls_b


# speedup vs baseline: 1.0445x; 1.0000x over previous
import functools

import jax
import jax.numpy as jnp
from jax.experimental import pallas as pl
from jax.experimental.pallas import tpu as pltpu

C_PAD = 128
S = 8
CSLOT = 16
NG = 8
N_CLS = 5


def _round_up(v, m):
    return (v + m - 1) // m * m


def _bdims(T, K0, stride0, poolk_a, pools_a, K3, poolk_b, pools_b):
    pad0_l = K0 // 2 + (K0 % 2) - 1
    pad0_r = K0 // 2
    Hp = T + pad0_l + pad0_r
    L0 = (Hp - K0) // stride0 + 1
    Ks0 = -(-K0 // stride0)
    L1 = (L0 - poolk_a) // pools_a + 1
    pad3 = K3 // 2 + (K3 % 2) - 1
    L2 = (L1 - poolk_b) // pools_b + 1
    return dict(K0=K0, stride0=stride0, pad0_l=pad0_l, pad0_r=pad0_r,
                L0=L0, Ks0=Ks0,
                poolk_a=poolk_a, pools_a=pools_a, L1=L1,
                K3=K3, pad3=pad3,
                poolk_b=poolk_b, pools_b=pools_b, L2=L2)


def _packed_kernel(xn_ref,
                   w01_ref, wm1_ref, bb1_ref,
                   w02_ref, wm2_ref, bb2_ref,
                   cls_ref, o_ref,
                   xts_g, buf0_g, buf1_g, buf2_g, *, d1, d2, tp, mp):
    f32 = jnp.float32

    ri = jax.lax.broadcasted_iota(jnp.int32, (S, C_PAD), 0)
    ci = jax.lax.broadcasted_iota(jnp.int32, (S, C_PAD), 1)
    ident = (ri == ci).astype(f32)
    t_len = xn_ref.shape[2]

    for g in range(NG):
        xts = xts_g[g]
        xts[pl.ds(0, mp), :] = jnp.zeros((mp, C_PAD), f32)
        xts[pl.ds(mp + t_len, tp - mp - t_len), :] = jnp.zeros(
            (tp - mp - t_len, C_PAD), f32)
        xts[pl.ds(mp, t_len), :] = jax.lax.dot_general(
            xn_ref[g], ident, (((0,), (0,)), ((), ())),
            preferred_element_type=f32)

    def run_branch(w0_ref, wm_ref, bb_ref, d):
        L0, st, off = d["L0"], d["stride0"], d["row_off"]
        poolk_a, pools_a, L1 = d["poolk_a"], d["pools_a"], d["L1"]
        K3, pad3 = d["K3"], d["pad3"]
        poolk_b, pools_b, L2 = d["poolk_b"], d["pools_b"], d["L2"]
        hi_pad = K3 - 1 - pad3

        for g in range(NG):
            for buf in (buf1_g[g], buf2_g[g]):
                buf[pl.ds(0, pad3), :] = jnp.zeros((pad3, C_PAD), f32)
                buf[pl.ds(pad3 + L1, hi_pad), :] = jnp.zeros(
                    (hi_pad, C_PAD), f32)

        Ks0 = d["Ks0"]
        Lph = L0 + Ks0 - 1
        for g in range(NG):
            xts = xts_g[g]
            xi = xts[pl.ds(off, Lph, stride=st), :]
            for r in range(1, st):
                xi = xi + jnp.roll(xts[pl.ds(off + r, Lph, stride=st), :],
                                   r * S, axis=1)
            acc = jnp.dot(xi[0:L0, :], w0_ref[0], preferred_element_type=f32)
            for ks in range(1, Ks0):
                acc = acc + jnp.dot(xi[ks:ks + L0, :], w0_ref[ks],
                                    preferred_element_type=f32)
            buf0_g[g][pl.ds(0, L0), :] = acc

        for g in range(NG):
            buf0 = buf0_g[g]
            pooled = buf0[pl.ds(0, L1, stride=pools_a), :]
            for r in range(1, poolk_a):
                pooled = jnp.maximum(pooled,
                                     buf0[pl.ds(r, L1, stride=pools_a), :])
            buf1_g[g][pl.ds(pad3, L1), :] = jnp.maximum(
                pooled + bb_ref[pl.ds(0, 1), :], 0.0)

        def conv_same(src_ref, layer):
            a = jnp.dot(src_ref[pl.ds(0, L1), :], wm_ref[layer, 0],
                        preferred_element_type=f32)
            for k in range(1, K3):
                a = a + jnp.dot(src_ref[pl.ds(k, L1), :], wm_ref[layer, k],
                                preferred_element_type=f32)
            return jnp.maximum(a + bb_ref[pl.ds(layer + 1, 1), :], 0.0)

        for g in range(NG):
            buf2_g[g][pl.ds(pad3, L1), :] = conv_same(buf1_g[g], 0)
        for g in range(NG):
            buf1_g[g][pl.ds(pad3, L1), :] = conv_same(buf2_g[g], 1)
        for g in range(NG):
            buf0_g[g][pl.ds(0, L1), :] = conv_same(buf1_g[g], 2)

        outs = []
        for g in range(NG):
            buf0 = buf0_g[g]
            out = buf0[pl.ds(0, L2, stride=pools_b), :]
            for r in range(1, poolk_b):
                out = jnp.maximum(out, buf0[pl.ds(r, L2, stride=pools_b), :])
            outs.append(out)
        return outs

    o1s = run_branch(w01_ref, wm1_ref, bb1_ref, d1)
    o2s = run_branch(w02_ref, wm2_ref, bb2_ref, d2)

    L2_1, L2_2 = d1["L2"], d2["L2"]
    row = jax.lax.broadcasted_iota(jnp.int32, (C_PAD, C_PAD), 0)
    col = jax.lax.broadcasted_iota(jnp.int32, (C_PAD, C_PAD), 1)
    sel = ((row // CSLOT) == col).astype(f32)
    for g in range(NG):
        rows = []
        for n in range(N_CLS):
            rows.append(
                jnp.sum(o1s[g] * cls_ref[n, pl.ds(0, L2_1), :],
                        axis=0, keepdims=True)
                + jnp.sum(o2s[g] * cls_ref[n, pl.ds(L2_1, L2_2), :],
                          axis=0, keepdims=True))
        rows.append(jnp.zeros((S - N_CLS, C_PAD), f32))
        vmat = jnp.concatenate(rows, axis=0)
        o_ref[g] = jnp.dot(vmat, sel, preferred_element_type=f32)


def _blockdiag(w):
    return jnp.kron(jnp.eye(S, dtype=w.dtype), w)


def kernel(x, b1_w0r, b1_wmid, b1_biases, b2_w0r, b2_wmid, b2_biases,
           cls_wperm, cls_b):
    T = x.shape[2]
    d1 = _bdims(T, 8, 2, 2, 2, 4, 2, 2)
    d2 = _bdims(T, 16, 4, 2, 2, 4, 2, 2)
    B = x.shape[0]
    G = B // S
    L2_sum = d1["L2"] + d2["L2"]
    xs = x[:, 0, :, 0]

    mp = max(d1["pad0_l"], d2["pad0_l"])
    d1["row_off"] = mp - d1["pad0_l"]
    d2["row_off"] = mp - d2["pad0_l"]
    need = max(d["row_off"] + d["K0"] + (d["L0"] - 1) * d["stride0"]
               for d in (d1, d2))
    TP = _round_up(need, 8)
    xng = xs.reshape(G, S, T)

    def _tap_weights(w0r, d):
        ks0, st = d["Ks0"], d["stride0"]
        w0k = w0r.reshape(ks0 * st, C_PAD)[:, :CSLOT].reshape(ks0, st, CSLOT)
        eye = jnp.eye(S, dtype=w0k.dtype)
        e = (w0k[:, :, None, None, :]
             * eye[None, None, :, :, None]).reshape(ks0, st * S, C_PAD)
        return jnp.pad(e, ((0, 0), (0, C_PAD - st * S), (0, 0)))

    w0b1 = _tap_weights(b1_w0r, d1)
    w0b2 = _tap_weights(b2_w0r, d2)

    wmb1 = jax.vmap(jax.vmap(_blockdiag))(b1_wmid[:, :, :CSLOT, :CSLOT])
    wmb2 = jax.vmap(jax.vmap(_blockdiag))(b2_wmid[:, :, :CSLOT, :CSLOT])
    bb1 = jnp.tile(b1_biases[:, :CSLOT], (1, S))
    bb2 = jnp.tile(b2_biases[:, :CSLOT], (1, S))

    wc = cls_wperm.reshape(L2_sum, C_PAD, N_CLS)[:, :CSLOT, :]
    wc = jnp.tile(jnp.transpose(wc, (2, 0, 1)), (1, 1, S))
    wc = jnp.pad(wc, ((0, S - N_CLS), (0, 0), (0, 0)))

    rows0 = _round_up(max(d1["L0"], d2["L0"]), 8)
    rows1 = _round_up(max(d1["L1"] + d1["K3"] - 1, d2["L1"] + d2["K3"] - 1), 8)

    kern = functools.partial(_packed_kernel, d1=d1, d2=d2, tp=TP, mp=mp)
    raw = pl.pallas_call(
        kern,
        out_shape=jax.ShapeDtypeStruct((G, S, C_PAD), jnp.float32),
        grid=(G // NG,),
        in_specs=[
            pl.BlockSpec((NG, S, T), lambda b: (b, 0, 0)),
            pl.BlockSpec((d1["Ks0"], C_PAD, C_PAD), lambda b: (0, 0, 0)),
            pl.BlockSpec((3, d1["K3"], C_PAD, C_PAD), lambda b: (0, 0, 0, 0)),
            pl.BlockSpec((4, C_PAD), lambda b: (0, 0)),
            pl.BlockSpec((d2["Ks0"], C_PAD, C_PAD), lambda b: (0, 0, 0)),
            pl.BlockSpec((3, d2["K3"], C_PAD, C_PAD), lambda b: (0, 0, 0, 0)),
            pl.BlockSpec((4, C_PAD), lambda b: (0, 0)),
            pl.BlockSpec((S, L2_sum, C_PAD), lambda b: (0, 0, 0)),
        ],
        out_specs=pl.BlockSpec((NG, S, C_PAD), lambda b: (b, 0, 0)),
        scratch_shapes=[
            [pltpu.VMEM((TP, C_PAD), jnp.float32) for _ in range(NG)],
            [pltpu.VMEM((rows0, C_PAD), jnp.float32) for _ in range(NG)],
            [pltpu.VMEM((rows1, C_PAD), jnp.float32) for _ in range(NG)],
            [pltpu.VMEM((rows1, C_PAD), jnp.float32) for _ in range(NG)],
        ],
        compiler_params=pltpu.CompilerParams(
            dimension_semantics=("parallel",)),
    )(xng, w0b1, wmb1, bb1, w0b2, wmb2, bb2, wc)

    logits = jnp.transpose(raw[:, :N_CLS, :S], (0, 2, 1)).reshape(B, N_CLS)
    return logits + cls_b
